# baseline (speedup 1.0000x reference)
import jax
import jax.numpy as jnp
from jax.experimental import pallas as pl
from jax.experimental.pallas import tpu as pltpu

_S = 2
_BM = 200


def _gconv_kernel(*refs):
    adj_refs = refs[:_S]
    x_ref, w_ref, b_ref, out_ref = refs[_S:]
    xb = x_ref[...].astype(jnp.bfloat16)
    for s in range(_S):
        t = jnp.dot(
            adj_refs[s][0].astype(jnp.bfloat16),
            xb,
            preferred_element_type=jnp.float32,
        )
        out_ref[s] = (
            jnp.dot(t, w_ref[...], preferred_element_type=jnp.float32)
            + b_ref[...]
        )


@jax.jit
def kernel(input, adj, weight, bias):
    n, d_in = input.shape
    d_out = weight.shape[1]
    m = adj.shape[0]
    ms = m // _S
    adj3 = adj.reshape(_S, ms, n)
    adj_specs = [
        pl.BlockSpec((1, _BM, n), lambda i, s=s: (s, i, 0)) for s in range(_S)
    ]
    out = pl.pallas_call(
        _gconv_kernel,
        grid=(ms // _BM,),
        in_specs=adj_specs + [
            pl.BlockSpec((n, d_in), lambda i: (0, 0)),
            pl.BlockSpec((d_in, d_out), lambda i: (0, 0)),
            pl.BlockSpec((1, d_out), lambda i: (0, 0)),
        ],
        out_specs=pl.BlockSpec((_S, _BM, d_out), lambda i: (0, i, 0)),
        out_shape=jax.ShapeDtypeStruct((_S, ms, d_out), jnp.float32),
        compiler_params=pltpu.CompilerParams(
            dimension_semantics=("arbitrary",),
            vmem_limit_bytes=120 * 1024 * 1024,
        ),
    )(*([adj3] * _S), input, weight, bias)
    return out.reshape(m, d_out)
